# Initial kernel scaffold; baseline (speedup 1.0000x reference)
import jax
import jax.numpy as jnp
from jax.experimental import pallas as pl


def kernel(x, edge_index, edge_weight, pos_src, W_pos, b_pos, attention, W_z, b_z, Lw_z, Lb_z, W_r, b_r, Lw_r, Lb_r, W_h, b_h, Lw_h, Lb_h, W_out, b_out):
    raise NotImplementedError("write your pallas kernel here")


# speedup vs baseline: 282.9188x; 282.9188x over previous
import functools

import jax
import jax.numpy as jnp
from jax import lax
from jax.experimental import pallas as pl
from jax.experimental.pallas import tpu as pltpu
from jax.experimental.pallas import tpu_sc as plsc

N = 100000
E = 3200000
FILTERS = 16
OUT_LEN = 14

NC = 2
NS = 16
NPT = 6272
N_PAD = NS * NPT
EPC = E // NC
EPW = E // (NC * NS)
CHUNK = 4000
NCHUNK = EPW // CHUNK
BN = 2048
NBLK = N_PAD // BN

_mesh = plsc.VectorSubcoreMesh(core_axis_name="c", subcore_axis_name="s")


@functools.partial(
    pl.kernel,
    out_type=jax.ShapeDtypeStruct((NC, N_PAD), jnp.float32),
    mesh=_mesh,
    scratch_types=[
        pltpu.VMEM((CHUNK,), jnp.int32),
        pltpu.VMEM((CHUNK,), jnp.float32),
        pltpu.VMEM_SHARED((N_PAD,), jnp.float32),
    ],
)
def _sc_degree(ei_hbm, ew_hbm, z_hbm, deg_out, didx, ewb, deg_sh):
    cid = lax.axis_index("c")
    sid = lax.axis_index("s")
    sl = pl.ds(sid * NPT, NPT)
    pltpu.sync_copy(z_hbm.at[sl], deg_sh.at[sl])
    plsc.subcore_barrier()
    ebase = cid * EPC + sid * EPW

    def body(k, carry):
        base = ebase + k * CHUNK
        pltpu.sync_copy(ei_hbm.at[pl.ds(E + base, CHUNK)], didx)
        pltpu.sync_copy(ew_hbm.at[pl.ds(base, CHUNK)], ewb)
        pltpu.sync_copy(ewb, deg_sh.at[didx], add=True)
        return carry

    lax.fori_loop(0, NCHUNK, body, 0)
    plsc.subcore_barrier()
    pltpu.sync_copy(deg_sh.at[sl], deg_out.at[cid, sl])


@functools.partial(
    pl.kernel,
    out_type=jax.ShapeDtypeStruct((NC, 2, N_PAD), jnp.float32),
    mesh=_mesh,
    scratch_types=[
        pltpu.VMEM((CHUNK,), jnp.int32),
        pltpu.VMEM((CHUNK,), jnp.int32),
        pltpu.VMEM((CHUNK,), jnp.float32),
        pltpu.VMEM((CHUNK,), jnp.float32),
        pltpu.VMEM((CHUNK,), jnp.float32),
        pltpu.VMEM_SHARED((N_PAD,), jnp.float32),
        pltpu.VMEM_SHARED((N_PAD,), jnp.float32),
        pltpu.VMEM_SHARED((N_PAD,), jnp.float32),
        pltpu.VMEM_SHARED((N_PAD,), jnp.float32),
    ],
)
def _sc_messages(ei_hbm, ew_hbm, y_hbm, z_hbm, acc_out,
                 sidx, didx, ewb, r0, r1, y0_sh, y1_sh, a0_sh, a1_sh):
    cid = lax.axis_index("c")
    sid = lax.axis_index("s")
    sl = pl.ds(sid * NPT, NPT)
    pltpu.sync_copy(y_hbm.at[0, sl], y0_sh.at[sl])
    pltpu.sync_copy(y_hbm.at[1, sl], y1_sh.at[sl])
    pltpu.sync_copy(z_hbm.at[sl], a0_sh.at[sl])
    pltpu.sync_copy(z_hbm.at[sl], a1_sh.at[sl])
    plsc.subcore_barrier()
    ebase = cid * EPC + sid * EPW

    def body(k, carry):
        base = ebase + k * CHUNK
        pltpu.sync_copy(ei_hbm.at[pl.ds(base, CHUNK)], sidx)
        pltpu.sync_copy(ei_hbm.at[pl.ds(E + base, CHUNK)], didx)
        pltpu.sync_copy(ew_hbm.at[pl.ds(base, CHUNK)], ewb)
        pltpu.sync_copy(y0_sh.at[sidx], r0)
        pltpu.sync_copy(y1_sh.at[sidx], r1)

        def mul(j, c):
            v = pl.ds(j * 16, 16)
            w = ewb[v]
            r0[v] = r0[v] * w
            r1[v] = r1[v] * w
            return c

        lax.fori_loop(0, CHUNK // 16, mul, 0)
        pltpu.sync_copy(r0, a0_sh.at[didx], add=True)
        pltpu.sync_copy(r1, a1_sh.at[didx], add=True)
        return carry

    lax.fori_loop(0, NCHUNK, body, 0)
    plsc.subcore_barrier()
    pltpu.sync_copy(a0_sh.at[sl], acc_out.at[cid, 0, sl])
    pltpu.sync_copy(a1_sh.at[sl], acc_out.at[cid, 1, sl])


def _tc_norm_body(xt_ref, post_ref, degp_ref, wpt_ref, bpos_ref, y_ref, dinv_ref):
    xb = jnp.nan_to_num(xt_ref[...])
    posb = post_ref[...]
    xf = xb + jnp.dot(wpt_ref[...], posb, preferred_element_type=jnp.float32)
    xf = xf + bpos_ref[...]
    degp = degp_ref[...]
    deg = degp[0:1, :] + degp[1:2, :] + 1.0
    dinv = jnp.where(deg > 0, lax.rsqrt(deg), 0.0)
    y_ref[...] = xf * dinv
    dinv_ref[...] = dinv


def _tc_norm(x_t, pos_t, deg_part, w_pos_t, b_pos_col):
    return pl.pallas_call(
        _tc_norm_body,
        grid=(NBLK,),
        in_specs=[
            pl.BlockSpec((2, BN), lambda i: (0, i)),
            pl.BlockSpec((9, BN), lambda i: (0, i)),
            pl.BlockSpec((2, BN), lambda i: (0, i)),
            pl.BlockSpec((2, 9), lambda i: (0, 0)),
            pl.BlockSpec((2, 1), lambda i: (0, 0)),
        ],
        out_specs=[
            pl.BlockSpec((2, BN), lambda i: (0, i)),
            pl.BlockSpec((1, BN), lambda i: (0, i)),
        ],
        out_shape=[
            jax.ShapeDtypeStruct((2, N_PAD), jnp.float32),
            jax.ShapeDtypeStruct((1, N_PAD), jnp.float32),
        ],
    )(x_t, pos_t, deg_part, w_pos_t, b_pos_col)


def _tc_out_body(sa_ref, y_ref, dinv_ref, consts_ref, wout_ref, bout_ref, out_ref):
    s = (sa_ref[...] + y_ref[...]) * dinv_ref[...]
    c = consts_ref[...]
    dn = (((0,), (0,)), ((), ()))
    H = jnp.zeros((BN, FILTERS), dtype=jnp.float32)
    for p in range(2):
        sp = s[p:p + 1, :]
        Asig = lax.dot_general(sp, c[0:1, :], dn, preferred_element_type=jnp.float32)
        Atan = lax.dot_general(sp, c[2:3, :], dn, preferred_element_type=jnp.float32)
        G = jax.nn.sigmoid(Asig + c[1:2, :])
        T = jnp.tanh(Atan + c[3:4, :])
        H = H + c[4 + p:5 + p, :] * (1.0 - G) * T
    h = jnp.maximum(H, 0.0)
    out_ref[...] = (jnp.dot(h, wout_ref[...], preferred_element_type=jnp.float32)
                    + bout_ref[...])


def _tc_out(sacc, y_t, dinv, consts, w_out, b_out_row):
    return pl.pallas_call(
        _tc_out_body,
        grid=(NBLK,),
        in_specs=[
            pl.BlockSpec((2, BN), lambda i: (0, i)),
            pl.BlockSpec((2, BN), lambda i: (0, i)),
            pl.BlockSpec((1, BN), lambda i: (0, i)),
            pl.BlockSpec((6, FILTERS), lambda i: (0, 0)),
            pl.BlockSpec((FILTERS, OUT_LEN), lambda i: (0, 0)),
            pl.BlockSpec((1, OUT_LEN), lambda i: (0, 0)),
        ],
        out_specs=pl.BlockSpec((BN, OUT_LEN), lambda i: (i, 0)),
        out_shape=jax.ShapeDtypeStruct((N_PAD, OUT_LEN), jnp.float32),
    )(sacc, y_t, dinv, consts, w_out, b_out_row)


def kernel(x, edge_index, edge_weight, pos_src, W_pos, b_pos, attention,
           W_z, b_z, Lw_z, Lb_z, W_r, b_r, Lw_r, Lb_r, W_h, b_h, Lw_h, Lb_h,
           W_out, b_out):
    pad = N_PAD - N
    x_t = jnp.pad(x, ((0, pad), (0, 0))).T
    pos_t = jnp.pad(pos_src, ((0, pad), (0, 0))).T
    zeros_n = jnp.zeros((N_PAD,), jnp.float32)

    az = (W_z @ Lw_z[:FILTERS])[0]
    cz = b_z @ Lw_z[:FILTERS] + Lb_z
    ah = (W_h @ Lw_h[:FILTERS])[0]
    ch = b_h @ Lw_h[:FILTERS] + Lb_h
    probs = jax.nn.softmax(attention, axis=0)
    consts = jnp.stack([
        az, cz, ah, ch,
        jnp.full((FILTERS,), 1.0, jnp.float32) * probs[0],
        jnp.full((FILTERS,), 1.0, jnp.float32) * probs[1],
    ])

    ei_flat = edge_index.reshape(2 * E)
    deg_part = _sc_degree(ei_flat, edge_weight, zeros_n)
    y_t, dinv = _tc_norm(x_t, pos_t, deg_part, W_pos.T, b_pos[:, None])
    acc_part = _sc_messages(ei_flat, edge_weight, y_t, zeros_n)
    sacc = acc_part[0] + acc_part[1]
    out_full = _tc_out(sacc, y_t, dinv, consts, W_out, b_out[None, :])
    return (out_full[:N],)
